# baseline (speedup 1.0000x reference)
import jax
import jax.numpy as jnp
from jax import lax
from jax.experimental import pallas as pl
from jax.experimental.pallas import tpu as pltpu
from jax.experimental.pallas import tpu_sc as plsc

N_ROWS = 160000
D = 256
N_SEG = 10000
NC = 2
NS = 16
NT = NC * NS
SPT = 320
SEG_PAD = NT * SPT
CH = 64
PAD = 16
BUF = CH + PAD
NV = D // 16


def _chunk_slices(r0, c):
    a = r0 + c * CH
    f = jnp.minimum((a // 8) * 8, N_ROWS - BUF)
    return a, f


def _body(x_hbm, b_hbm, q_hbm, cuts_hbm, out_hbm,
          xb0, bid0, xb1, bid1, qv_ref, cuts_ref, acc_ref, den_ref, esc_ref,
          sx0, sb0, sx1, sb1):
    wid = lax.axis_index("s") * NC + lax.axis_index("c")
    pltpu.sync_copy(q_hbm, qv_ref)
    pltpu.sync_copy(cuts_hbm, cuts_ref)
    cv = cuts_ref[pl.ds(wid, 16)]
    r0 = cv[0]
    r1 = cv[1]
    s0 = wid * SPT

    zf = jnp.zeros((16,), jnp.float32)

    @plsc.parallel_loop(0, SPT, 1, unroll=4)
    def _(si):
        for v in range(NV):
            acc_ref[si, pl.ds(16 * v, 16)] = zf

    @plsc.parallel_loop(0, SPT, 1, unroll=8)
    def _(si):
        den_ref[pl.ds(si * 16, 16)] = zf

    n_chunks = (r1 - r0 + CH - 1) // CH
    qs = [qv_ref[pl.ds(16 * v, 16)] for v in range(NV)]
    iota16 = lax.iota(jnp.int32, 16)

    def issue(xb, bid, sx, sb, c):
        _, f = _chunk_slices(r0, c)
        pltpu.async_copy(x_hbm.at[pl.ds(f, BUF)], xb, sx)
        pltpu.async_copy(b_hbm.at[pl.ds(f, BUF)], bid.at[pl.ds(0, BUF)], sb)

    def wait(xb, bid, sx, sb, c):
        _, f = _chunk_slices(r0, c)
        pltpu.make_async_copy(x_hbm.at[pl.ds(f, BUF)], xb, sx).wait()
        pltpu.make_async_copy(b_hbm.at[pl.ds(f, BUF)], bid.at[pl.ds(0, BUF)], sb).wait()

    def process(xb, bid, c):
        a, f = _chunk_slices(r0, c)
        off = a - f

        @plsc.parallel_loop(0, CH, 1, unroll=4)
        def _(r):
            valid = (a + r) < r1
            idx = jnp.minimum(off + r, BUF - 1)
            xs = [xb[idx, pl.ds(16 * v, 16)] for v in range(NV)]
            accs = [xs[v] * qs[v] for v in range(4)]
            for v in range(4, NV):
                accs[v % 4] = accs[v % 4] + xs[v] * qs[v]
            dv = (accs[0] + accs[1]) + (accs[2] + accs[3])
            for k in (8, 4, 2, 1):
                rot = dv.at[(iota16 + k) % 16].get(mode="promise_in_bounds")
                dv = dv + rot
            vf = jnp.where(valid, jnp.float32(1), jnp.float32(0))
            esc_ref[pl.ds(r * 16, 16)] = jnp.exp(dv) * jnp.full((16,), vf)

        def group_body(g, _):
            rowids = jnp.minimum(off + g * 16 + iota16, BUF - 1)
            ev16 = plsc.load_gather(esc_ref, [g * 256 + iota16 * 17])
            bvec = plsc.load_gather(bid, [rowids])
            lid16 = jnp.minimum(jnp.maximum(bvec - s0, 0), SPT - 1)
            plsc.addupdate_scatter(den_ref, [lid16 * 16 + iota16], ev16)

            @plsc.parallel_loop(0, D, 1, unroll=16)
            def _(d):
                dsplat = jnp.full((16,), d, jnp.int32)
                xv = plsc.load_gather(xb, [rowids, dsplat])
                plsc.addupdate_scatter(acc_ref, [lid16, dsplat], xv * ev16)

            return 0

        lax.fori_loop(0, CH // 16, group_body, 0)

    @pl.when(n_chunks > 0)
    def _():
        issue(xb0, bid0, sx0, sb0, 0)

    def pair_body(p, _):
        c0 = 2 * p
        c1 = c0 + 1
        wait(xb0, bid0, sx0, sb0, c0)

        @pl.when(c1 < n_chunks)
        def _():
            issue(xb1, bid1, sx1, sb1, c1)

        process(xb0, bid0, c0)

        @pl.when(c1 < n_chunks)
        def _():
            wait(xb1, bid1, sx1, sb1, c1)

            @pl.when(c1 + 1 < n_chunks)
            def _():
                issue(xb0, bid0, sx0, sb0, c1 + 1)

            process(xb1, bid1, c1)

        return 0

    lax.fori_loop(0, (n_chunks + 1) // 2, pair_body, 0)

    @plsc.parallel_loop(0, SPT, 1, unroll=2)
    def _(si):
        dvec = den_ref[pl.ds(si * 16, 16)]
        for k in (8, 4, 2, 1):
            dvec = dvec + dvec.at[(iota16 + k) % 16].get(mode="promise_in_bounds")
        dsafe = jnp.maximum(dvec, jnp.float32(1e-37))
        for v in range(NV):
            acc_ref[si, pl.ds(16 * v, 16)] = acc_ref[si, pl.ds(16 * v, 16)] / dsafe

    pltpu.sync_copy(acc_ref, out_hbm.at[pl.ds(s0, SPT)])


def kernel(x, batch, query):
    batch32 = batch.astype(jnp.int32)
    cut_ids = jnp.minimum(jnp.arange(NT + 1, dtype=jnp.int32) * SPT, N_SEG)
    cuts = jnp.searchsorted(batch32, cut_ids, side="left").astype(jnp.int32)
    cuts = jnp.pad(cuts, (0, 64 - (NT + 1)))

    mesh = plsc.VectorSubcoreMesh(core_axis_name="c", subcore_axis_name="s")
    fn = pl.kernel(
        _body,
        out_type=jax.ShapeDtypeStruct((SEG_PAD, D), jnp.float32),
        mesh=mesh,
        compiler_params=pltpu.CompilerParams(needs_layout_passes=False),
        scratch_types=[
            pltpu.VMEM((BUF, D), jnp.float32),
            pltpu.VMEM((BUF + 16,), jnp.int32),
            pltpu.VMEM((BUF, D), jnp.float32),
            pltpu.VMEM((BUF + 16,), jnp.int32),
            pltpu.VMEM((D,), jnp.float32),
            pltpu.VMEM((64,), jnp.int32),
            pltpu.VMEM((SPT, D), jnp.float32),
            pltpu.VMEM((SPT * 16,), jnp.float32),
            pltpu.VMEM((CH * 16,), jnp.float32),
            pltpu.SemaphoreType.DMA,
            pltpu.SemaphoreType.DMA,
            pltpu.SemaphoreType.DMA,
            pltpu.SemaphoreType.DMA,
        ],
    )
    out = fn(x, batch32, query, cuts)
    return out[:N_SEG]


# speedup vs baseline: 7.5702x; 7.5702x over previous
import jax
import jax.numpy as jnp
from jax import lax
from jax.experimental import pallas as pl
from jax.experimental.pallas import tpu as pltpu
from jax.experimental.pallas import tpu_sc as plsc

N_ROWS = 160000
D = 256
N_SEG = 10000
NC = 2
NS = 16
NT = NC * NS
SPT = 320
SEG_PAD = NT * SPT
CH = 64
PAD = 16
BUF = CH + PAD
NV = D // 16


def _chunk_slices(r0, c):
    a = r0 + c * CH
    f = jnp.minimum((a // 8) * 8, N_ROWS - BUF)
    return a, f


def _body(x_hbm, b_hbm, q_hbm, cuts_hbm, out_hbm,
          xb0, bid0, xb1, bid1, qv_ref, cuts_ref, acc_ref, den_ref, esc_ref,
          sx0, sb0, sx1, sb1):
    wid = lax.axis_index("s") * NC + lax.axis_index("c")
    pltpu.sync_copy(q_hbm, qv_ref)
    pltpu.sync_copy(cuts_hbm, cuts_ref)
    cv = cuts_ref[pl.ds(wid, 16)]
    r0 = cv[0]
    r1 = cv[1]
    s0 = wid * SPT

    zf = jnp.zeros((16,), jnp.float32)

    @plsc.parallel_loop(0, SPT, 1, unroll=4)
    def _(si):
        for v in range(NV):
            acc_ref[si, pl.ds(16 * v, 16)] = zf

    @plsc.parallel_loop(0, SPT, 1, unroll=8)
    def _(si):
        den_ref[pl.ds(si * 16, 16)] = zf

    n_chunks = (r1 - r0 + CH - 1) // CH
    qs = [qv_ref[pl.ds(16 * v, 16)] for v in range(NV)]
    iota16 = lax.iota(jnp.int32, 16)

    def issue(xb, bid, sx, sb, c):
        _, f = _chunk_slices(r0, c)
        pltpu.async_copy(x_hbm.at[pl.ds(f, BUF)], xb, sx)
        pltpu.async_copy(b_hbm.at[pl.ds(f, BUF)], bid.at[pl.ds(0, BUF)], sb)

    def wait(xb, bid, sx, sb, c):
        _, f = _chunk_slices(r0, c)
        pltpu.make_async_copy(x_hbm.at[pl.ds(f, BUF)], xb, sx).wait()
        pltpu.make_async_copy(b_hbm.at[pl.ds(f, BUF)], bid.at[pl.ds(0, BUF)], sb).wait()

    def process(xb, bid, c):
        a, f = _chunk_slices(r0, c)
        off = a - f

        @plsc.parallel_loop(0, CH, 1, unroll=2)
        def _(r):
            valid = (a + r) < r1
            idx = jnp.minimum(off + r, BUF - 1)
            xs = [xb[idx, pl.ds(16 * v, 16)] for v in range(NV)]
            accs = [xs[v] * qs[v] for v in range(4)]
            for v in range(4, NV):
                accs[v % 4] = accs[v % 4] + xs[v] * qs[v]
            dv = (accs[0] + accs[1]) + (accs[2] + accs[3])
            for k in (8, 4, 2, 1):
                rot = dv.at[(iota16 + k) % 16].get(mode="promise_in_bounds")
                dv = dv + rot
            vf = jnp.where(valid, jnp.float32(1), jnp.float32(0))
            esc_ref[pl.ds(r * 16, 16)] = jnp.exp(dv) * jnp.full((16,), vf)

        @plsc.parallel_loop(0, CH, 1, unroll=2)
        def _(r):
            valid = (a + r) < r1
            idx = jnp.minimum(off + r, BUF - 1)
            ev = esc_ref[pl.ds(r * 16, 16)]
            bv = bid[pl.ds(idx, 16)]
            lid = jnp.where(valid, bv[0] - s0, 0)
            for v in range(NV):
                plsc.addupdate(acc_ref.at[lid, pl.ds(16 * v, 16)],
                               xb[idx, pl.ds(16 * v, 16)] * ev)
            plsc.addupdate(den_ref.at[pl.ds(lid * 16, 16)], ev)

    @pl.when(n_chunks > 0)
    def _():
        issue(xb0, bid0, sx0, sb0, 0)

    def pair_body(p, _):
        c0 = 2 * p
        c1 = c0 + 1
        wait(xb0, bid0, sx0, sb0, c0)

        @pl.when(c1 < n_chunks)
        def _():
            issue(xb1, bid1, sx1, sb1, c1)

        process(xb0, bid0, c0)

        @pl.when(c1 < n_chunks)
        def _():
            wait(xb1, bid1, sx1, sb1, c1)

            @pl.when(c1 + 1 < n_chunks)
            def _():
                issue(xb0, bid0, sx0, sb0, c1 + 1)

            process(xb1, bid1, c1)

        return 0

    lax.fori_loop(0, (n_chunks + 1) // 2, pair_body, 0)

    @plsc.parallel_loop(0, SPT, 1, unroll=2)
    def _(si):
        dvec = den_ref[pl.ds(si * 16, 16)]
        dsafe = jnp.maximum(dvec, jnp.float32(1e-37))
        for v in range(NV):
            acc_ref[si, pl.ds(16 * v, 16)] = acc_ref[si, pl.ds(16 * v, 16)] / dsafe

    pltpu.sync_copy(acc_ref, out_hbm.at[pl.ds(s0, SPT)])


def kernel(x, batch, query):
    batch32 = batch.astype(jnp.int32)
    cut_ids = jnp.minimum(jnp.arange(NT + 1, dtype=jnp.int32) * SPT, N_SEG)
    cuts = jnp.searchsorted(batch32, cut_ids, side="left").astype(jnp.int32)
    cuts = jnp.pad(cuts, (0, 64 - (NT + 1)))

    mesh = plsc.VectorSubcoreMesh(core_axis_name="c", subcore_axis_name="s")
    fn = pl.kernel(
        _body,
        out_type=jax.ShapeDtypeStruct((SEG_PAD, D), jnp.float32),
        mesh=mesh,
        scratch_types=[
            pltpu.VMEM((BUF, D), jnp.float32),
            pltpu.VMEM((BUF + 16,), jnp.int32),
            pltpu.VMEM((BUF, D), jnp.float32),
            pltpu.VMEM((BUF + 16,), jnp.int32),
            pltpu.VMEM((D,), jnp.float32),
            pltpu.VMEM((64,), jnp.int32),
            pltpu.VMEM((SPT, D), jnp.float32),
            pltpu.VMEM((SPT * 16,), jnp.float32),
            pltpu.VMEM((CH * 16,), jnp.float32),
            pltpu.SemaphoreType.DMA,
            pltpu.SemaphoreType.DMA,
            pltpu.SemaphoreType.DMA,
            pltpu.SemaphoreType.DMA,
        ],
    )
    out = fn(x, batch32, query, cuts)
    return out[:N_SEG]


# speedup vs baseline: 7.6041x; 1.0045x over previous
import jax
import jax.numpy as jnp
from jax import lax
from jax.experimental import pallas as pl
from jax.experimental.pallas import tpu as pltpu
from jax.experimental.pallas import tpu_sc as plsc

N_ROWS = 160000
D = 256
N_SEG = 10000
NC = 2
NS = 16
NT = NC * NS
SPT = 320
SEG_PAD = NT * SPT
CH = 64
PAD = 16
BUF = CH + PAD
NV = D // 16


def _chunk_slices(r0, c):
    a = r0 + c * CH
    f = jnp.minimum((a // 8) * 8, N_ROWS - BUF)
    return a, f


def _body(x_hbm, b_hbm, q_hbm, cuts_hbm, out_hbm,
          xb0, bid0, xb1, bid1, qv_ref, cuts_ref, acc_ref, den_ref, esc_ref,
          sx0, sb0, sx1, sb1):
    wid = lax.axis_index("s") * NC + lax.axis_index("c")
    pltpu.sync_copy(q_hbm, qv_ref)
    pltpu.sync_copy(cuts_hbm, cuts_ref)
    cv = cuts_ref[pl.ds(wid, 16)]
    r0 = cv[0]
    r1 = cv[1]
    s0 = wid * SPT

    zf = jnp.zeros((16,), jnp.float32)

    @plsc.parallel_loop(0, SPT, 1, unroll=4)
    def _(si):
        for v in range(NV):
            acc_ref[si, pl.ds(16 * v, 16)] = zf

    @plsc.parallel_loop(0, SPT, 1, unroll=8)
    def _(si):
        den_ref[pl.ds(si * 16, 16)] = zf

    n_chunks = (r1 - r0 + CH - 1) // CH
    qs = [qv_ref[pl.ds(16 * v, 16)] for v in range(NV)]
    iota16 = lax.iota(jnp.int32, 16)

    def issue(xb, bid, sx, sb, c):
        _, f = _chunk_slices(r0, c)
        pltpu.async_copy(x_hbm.at[pl.ds(f, BUF)], xb, sx)
        pltpu.async_copy(b_hbm.at[pl.ds(f, BUF)], bid.at[pl.ds(0, BUF)], sb)

    def wait(xb, bid, sx, sb, c):
        _, f = _chunk_slices(r0, c)
        pltpu.make_async_copy(x_hbm.at[pl.ds(f, BUF)], xb, sx).wait()
        pltpu.make_async_copy(b_hbm.at[pl.ds(f, BUF)], bid.at[pl.ds(0, BUF)], sb).wait()

    def process(xb, bid, c):
        a, f = _chunk_slices(r0, c)
        off = a - f

        @plsc.parallel_loop(0, CH, 1, unroll=2)
        def _(r):
            valid = (a + r) < r1
            idx = jnp.minimum(off + r, BUF - 1)
            xs = [xb[idx, pl.ds(16 * v, 16)] for v in range(NV)]
            accs = [xs[v] * qs[v] for v in range(4)]
            for v in range(4, NV):
                accs[v % 4] = accs[v % 4] + xs[v] * qs[v]
            dv = (accs[0] + accs[1]) + (accs[2] + accs[3])
            for k in (8, 4, 2, 1):
                rot = dv.at[(iota16 + k) % 16].get(mode="promise_in_bounds")
                dv = dv + rot
            vf = jnp.where(valid, jnp.float32(1), jnp.float32(0))
            esc_ref[pl.ds(r * 16, 16)] = jnp.exp(dv) * jnp.full((16,), vf)

        @plsc.parallel_loop(0, CH, 1, unroll=2)
        def _(r):
            valid = (a + r) < r1
            idx = jnp.minimum(off + r, BUF - 1)
            ev = esc_ref[pl.ds(r * 16, 16)]
            bv = bid[pl.ds(idx, 16)]
            lid = jnp.where(valid, bv[0] - s0, 0)
            for v in range(NV):
                plsc.addupdate(acc_ref.at[lid, pl.ds(16 * v, 16)],
                               xb[idx, pl.ds(16 * v, 16)] * ev)
            plsc.addupdate(den_ref.at[pl.ds(lid * 16, 16)], ev)

    @pl.when(n_chunks > 0)
    def _():
        issue(xb0, bid0, sx0, sb0, 0)

    def pair_body(p, _):
        c0 = 2 * p
        c1 = c0 + 1
        wait(xb0, bid0, sx0, sb0, c0)

        @pl.when(c1 < n_chunks)
        def _():
            issue(xb1, bid1, sx1, sb1, c1)

        process(xb0, bid0, c0)

        @pl.when(c1 < n_chunks)
        def _():
            wait(xb1, bid1, sx1, sb1, c1)

            @pl.when(c1 + 1 < n_chunks)
            def _():
                issue(xb0, bid0, sx0, sb0, c1 + 1)

            process(xb1, bid1, c1)

        return 0

    lax.fori_loop(0, (n_chunks + 1) // 2, pair_body, 0)

    @plsc.parallel_loop(0, SPT, 1, unroll=2)
    def _(si):
        dvec = den_ref[pl.ds(si * 16, 16)]
        dsafe = jnp.maximum(dvec, jnp.float32(1e-37))
        for v in range(NV):
            acc_ref[si, pl.ds(16 * v, 16)] = acc_ref[si, pl.ds(16 * v, 16)] / dsafe

    pltpu.sync_copy(acc_ref, out_hbm.at[pl.ds(s0, SPT)])


def kernel(x, batch, query):
    batch32 = batch.astype(jnp.int32)
    cut_ids = jnp.minimum(jnp.arange(NT + 1, dtype=jnp.int32) * SPT, N_SEG)
    cuts = jnp.searchsorted(batch32, cut_ids, side="left").astype(jnp.int32)
    cuts = jnp.pad(cuts, (0, 64 - (NT + 1)))

    mesh = plsc.VectorSubcoreMesh(core_axis_name="c", subcore_axis_name="s")
    fn = pl.kernel(
        _body,
        out_type=jax.ShapeDtypeStruct((SEG_PAD, D), jnp.float32),
        mesh=mesh,
        compiler_params=pltpu.CompilerParams(needs_layout_passes=False),
        scratch_types=[
            pltpu.VMEM((BUF, D), jnp.float32),
            pltpu.VMEM((BUF + 16,), jnp.int32),
            pltpu.VMEM((BUF, D), jnp.float32),
            pltpu.VMEM((BUF + 16,), jnp.int32),
            pltpu.VMEM((D,), jnp.float32),
            pltpu.VMEM((64,), jnp.int32),
            pltpu.VMEM((SPT, D), jnp.float32),
            pltpu.VMEM((SPT * 16,), jnp.float32),
            pltpu.VMEM((CH * 16,), jnp.float32),
            pltpu.SemaphoreType.DMA,
            pltpu.SemaphoreType.DMA,
            pltpu.SemaphoreType.DMA,
            pltpu.SemaphoreType.DMA,
        ],
    )
    out = fn(x, batch32, query, cuts)
    return out[:N_SEG]
